# Initial kernel scaffold; baseline (speedup 1.0000x reference)
import jax
import jax.numpy as jnp
from jax.experimental import pallas as pl


def kernel(x, conv_w, conv_b, fc1_w, fc1_b, fc2_w, fc2_b):
    raise NotImplementedError("write your pallas kernel here")


# speedup vs baseline: 4.8461x; 4.8461x over previous
import functools

import jax
import jax.numpy as jnp
from jax.experimental import pallas as pl
from jax.experimental.pallas import tpu as pltpu

B, C, T, H, W = 4, 3, 64, 224, 224
TBLK = 8
NTB = T // TBLK
HW = H * W


def _sum_body(x_ref, cw_ref, cb_ref, out_ref):
    b = pl.program_id(0)
    tb = pl.program_id(1)
    w0 = cw_ref[0, 0]
    w1 = cw_ref[0, 1]
    w2 = cw_ref[0, 2]
    c0 = cb_ref[0]
    v = x_ref[0, 0] * w0 + x_ref[0, 1] * w1 + x_ref[0, 2] * w2 + c0
    v = jnp.maximum(v, 0.0)
    sums = jnp.sum(v, axis=(1, 2))
    row = jax.lax.broadcasted_iota(jnp.int32, (1, NTB, TBLK), 1)
    bcast = jnp.broadcast_to(sums[None, None, :], (1, NTB, TBLK))
    out_ref[...] = jnp.where(row == tb, bcast, out_ref[...])


def _frame_sums(x, conv_w, conv_b):
    return pl.pallas_call(
        _sum_body,
        grid=(B, NTB),
        in_specs=[
            pl.BlockSpec((1, C, TBLK, H, W), lambda b, tb: (b, 0, tb, 0, 0)),
            pl.BlockSpec(memory_space=pltpu.SMEM),
            pl.BlockSpec(memory_space=pltpu.SMEM),
        ],
        out_specs=pl.BlockSpec((1, NTB, TBLK), lambda b, tb: (b, 0, 0)),
        out_shape=jax.ShapeDtypeStruct((B, NTB, TBLK), jnp.float32),
    )(x, conv_w, conv_b)


def _score_body(y_ref, f1w_ref, f1b_ref, f2w_ref, f2b_ref, idx_ref):
    y = y_ref[...] * (1.0 / HW)
    z = jax.lax.dot_general(y, f1w_ref[...], (((1,), (1,)), ((), ())),
                            preferred_element_type=jnp.float32)
    z = jnp.maximum(z + f1b_ref[...][None, :], 0.0)
    lg = jax.lax.dot_general(z, f2w_ref[...], (((1,), (1,)), ((), ())),
                             preferred_element_type=jnp.float32)
    lg = lg + f2b_ref[...][None, :]
    s = 1.0 / (1.0 + jnp.exp(-lg))

    iota_t = jax.lax.broadcasted_iota(jnp.int32, (B, T), 1)
    iota_o = jax.lax.broadcasted_iota(jnp.int32, (B, 8), 1)
    out = jnp.zeros((B, 8), jnp.int32)
    big = jnp.float32(jnp.inf)

    work = s
    for k in range(4):
        m = jnp.max(work, axis=1, keepdims=True)
        a = jnp.min(jnp.where(work == m, iota_t, T), axis=1, keepdims=True)
        out = jnp.where(iota_o == k, a, out)
        work = jnp.where(iota_t == a, -big, work)
    work = s
    for k in range(4):
        m = jnp.min(work, axis=1, keepdims=True)
        a = jnp.min(jnp.where(work == m, iota_t, T), axis=1, keepdims=True)
        out = jnp.where(iota_o == (4 + k), a, out)
        work = jnp.where(iota_t == a, big, work)
    idx_ref[...] = out


def _score(y, fc1_w, fc1_b, fc2_w, fc2_b):
    return pl.pallas_call(
        _score_body,
        in_specs=[pl.BlockSpec(memory_space=pltpu.VMEM)] * 5,
        out_specs=pl.BlockSpec(memory_space=pltpu.VMEM),
        out_shape=jax.ShapeDtypeStruct((B, 8), jnp.int32),
    )(y, fc1_w, fc1_b, fc2_w, fc2_b)


def _gather_body(idx_ref, x_ref, cw_ref, cb_ref, out_ref):
    w0 = cw_ref[0, 0]
    w1 = cw_ref[0, 1]
    w2 = cw_ref[0, 2]
    c0 = cb_ref[0]
    v = x_ref[0, 0, 0] * w0 + x_ref[0, 1, 0] * w1 + x_ref[0, 2, 0] * w2 + c0
    out_ref[0, 0, 0] = jnp.maximum(v, 0.0)


def _gather(idx, x, conv_w, conv_b):
    grid_spec = pltpu.PrefetchScalarGridSpec(
        num_scalar_prefetch=1,
        grid=(B, 8),
        in_specs=[
            pl.BlockSpec((1, C, 1, H, W),
                         lambda b, j, iref: (b, 0, iref[b, j], 0, 0)),
            pl.BlockSpec(memory_space=pltpu.SMEM),
            pl.BlockSpec(memory_space=pltpu.SMEM),
        ],
        out_specs=pl.BlockSpec((1, 1, 1, H, W),
                               lambda b, j, iref: (b, 0, j, 0, 0)),
    )
    return pl.pallas_call(
        _gather_body,
        grid_spec=grid_spec,
        out_shape=jax.ShapeDtypeStruct((B, 1, 8, H, W), jnp.float32),
    )(idx, x, conv_w, conv_b)


def kernel(x, conv_w, conv_b, fc1_w, fc1_b, fc2_w, fc2_b):
    part = _frame_sums(x, conv_w, conv_b)
    y = part.reshape(B, T)
    idx = _score(y, fc1_w, fc1_b, fc2_w, fc2_b)
    return _gather(idx, x, conv_w, conv_b)
